# Initial kernel scaffold; baseline (speedup 1.0000x reference)
import jax
import jax.numpy as jnp
from jax.experimental import pallas as pl


def kernel(queries, keys, Pi, high_centroids, low_centroids, S_high, S_low):
    raise NotImplementedError("write your pallas kernel here")


# speedup vs baseline: 6.8572x; 6.8572x over previous
import functools
import math

import jax
import jax.numpy as jnp
from jax.experimental import pallas as pl
from jax.experimental.pallas import tpu as pltpu

D = 256
NH = 128
NL = 128
BQ = 4096
BK = 4096
QBLK = 512
SCALE = math.sqrt(math.pi / 2.0) / 128.0

def _dot(a, b, dims):
    return jax.lax.dot_general(a.astype(jnp.bfloat16),
                               b.astype(jnp.bfloat16), (dims, ((), ())),
                               preferred_element_type=jnp.float32)


def _nearest(y, c_ref, n):
    c0 = c_ref[0]
    best_c = jnp.full_like(y, c0)
    best_d = (y - c0) ** 2
    for j in range(1, n):
        cj = c_ref[j]
        dj = (y - cj) ** 2
        upd = dj < best_d
        best_c = jnp.where(upd, cj, best_c)
        best_d = jnp.where(upd, dj, best_d)
    return best_c


def _body(ch_ref, cl_ref, q_ref, k_ref, pi_ref, sh_ref, sl_ref, out_ref,
          k2_ref):
    @pl.when(pl.program_id(0) == 0)
    def _build_k2():
        keys = k_ref[...]
        vn = jnp.sqrt(jnp.sum(keys * keys, axis=1, keepdims=True))
        kn = keys / (vn + 1e-8)
        parts = []
        for (lo, n_ch, c_ref, n_cent, s_ref) in (
                (0, NH, ch_ref, 4, sh_ref),
                (NH, NL, cl_ref, 2, sl_ref)):
            y = _dot(kn, pi_ref[lo:lo + n_ch, :], (((1,), (1,))))
            y_mse = _nearest(y, c_ref, n_cent)
            resid = y - y_mse
            rnorm = jnp.sqrt(jnp.sum(resid * resid, axis=1, keepdims=True))
            proj = _dot(resid, s_ref[...], (((1,), (1,))))
            signs = jnp.where(proj >= 0.0, 1.0, -1.0)
            corr = _dot(signs, s_ref[...], (((1,), (0,))))
            keff_g = vn * (y_mse + (SCALE * rnorm) * corr)
            parts.append(_dot(keff_g, pi_ref[lo:lo + n_ch, :],
                              (((1,), (0,)))))
        k2_ref[...] = parts[0] + parts[1]

    out_ref[...] = _dot(q_ref[...], k2_ref[...], (((1,), (1,))))


@jax.jit
def kernel(queries, keys, Pi, high_centroids, low_centroids, S_high, S_low):
    grid = BQ // QBLK
    est = pl.pallas_call(
        _body,
        grid=(grid,),
        in_specs=[
            pl.BlockSpec(memory_space=pltpu.SMEM),
            pl.BlockSpec(memory_space=pltpu.SMEM),
            pl.BlockSpec((QBLK, D), lambda i: (i, 0)),
            pl.BlockSpec((BK, D), lambda i: (0, 0)),
            pl.BlockSpec((D, D), lambda i: (0, 0)),
            pl.BlockSpec((NH, NH), lambda i: (0, 0)),
            pl.BlockSpec((NL, NL), lambda i: (0, 0)),
        ],
        out_specs=pl.BlockSpec((QBLK, BK), lambda i: (i, 0)),
        out_shape=jax.ShapeDtypeStruct((BQ, BK), jnp.float32),
        scratch_shapes=[pltpu.VMEM((BK, D), jnp.float32)],
    )(high_centroids, low_centroids, queries, keys, Pi, S_high, S_low)
    return est
